# Initial kernel scaffold; baseline (speedup 1.0000x reference)
import jax
import jax.numpy as jnp
from jax.experimental import pallas as pl


def kernel(x, edge_index, Wl0, Wr0, b0, Wl1, Wr1, b1, Wl2, Wr2, b2, R):
    raise NotImplementedError("write your pallas kernel here")


# speedup vs baseline: 6.3607x; 6.3607x over previous
import functools

import jax
import jax.numpy as jnp
from jax import lax
from jax.experimental import pallas as pl
from jax.experimental.pallas import tpu as pltpu
from jax.experimental.pallas import tpu_sc as plsc

N = 10000
E = 320000
CHUNK = 128
NWORK = 32
EPW = E // NWORK
MAIN_CHUNKS = EPW // CHUNK
TAIL = EPW - MAIN_CHUNKS * CHUNK
SUB_ROWS = 632
LAST_ROWS = N - 15 * SUB_ROWS
BM = 2000
GRID = N // BM


def _copy_rowrange(sid, src_ref, dst_ref):
    @pl.when(sid < 15)
    def _():
        pltpu.sync_copy(src_ref.at[pl.ds(sid * SUB_ROWS, SUB_ROWS)],
                        dst_ref.at[pl.ds(sid * SUB_ROWS, SUB_ROWS)])

    @pl.when(sid == 15)
    def _():
        pltpu.sync_copy(src_ref.at[pl.ds(15 * SUB_ROWS, LAST_ROWS)],
                        dst_ref.at[pl.ds(15 * SUB_ROWS, LAST_ROWS)])


def _make_sc_agg(W: int, with_deg: bool, src_spmem: bool = False):
    mesh = plsc.VectorSubcoreMesh(core_axis_name="c", subcore_axis_name="s")
    out_type = [jax.ShapeDtypeStruct((2, N, W), jnp.float32)]
    scratch = [
        pltpu.VMEM((CHUNK,), jnp.int32),
        pltpu.VMEM((CHUNK,), jnp.int32),
        pltpu.VMEM((TAIL,), jnp.int32),
        pltpu.VMEM((TAIL,), jnp.int32),
        pltpu.VMEM((CHUNK, W), jnp.float32),
        pltpu.VMEM((TAIL, W), jnp.float32),
        pltpu.VMEM_SHARED((N, W), jnp.float32),
        pltpu.SemaphoreType.DMA,
    ]
    if src_spmem:
        scratch.append(pltpu.VMEM_SHARED((N, W), jnp.float32))
    if with_deg:
        out_type.append(jax.ShapeDtypeStruct((2, N), jnp.float32))
        scratch += [
            pltpu.VMEM((CHUNK,), jnp.float32),
            pltpu.VMEM((TAIL,), jnp.float32),
            pltpu.VMEM_SHARED((N,), jnp.float32),
        ]

    def body(z, srcm, dstm, zero2, *rest):
        if with_deg:
            (zero1, part, degp, srcv, dstv, srct, dstt, rows, rowst, acc,
             sem, onesv, onest, dacc) = rest
        elif src_spmem:
            part, srcv, dstv, srct, dstt, rows, rowst, acc, sem, zsrc = rest
        else:
            part, srcv, dstv, srct, dstt, rows, rowst, acc, sem = rest
        cid = lax.axis_index("c")
        sid = lax.axis_index("s")
        wid = sid * 2 + cid

        _copy_rowrange(sid, zero2, acc)
        if src_spmem:
            _copy_rowrange(sid, z, zsrc)
            gsrc = zsrc
        else:
            gsrc = z
        if with_deg:
            @pl.when(sid == 0)
            def _():
                pltpu.sync_copy(zero1, dacc)
            for i in range(CHUNK // 16):
                onesv[pl.ds(i * 16, 16)] = jnp.ones((16,), jnp.float32)
            onest[...] = jnp.ones((TAIL,), jnp.float32)
        plsc.subcore_barrier()

        base = wid * EPW

        def step(g, carry):
            off = base + g * CHUNK
            pltpu.sync_copy(srcm.at[pl.ds(off, CHUNK)], srcv)
            pltpu.sync_copy(dstm.at[pl.ds(off, CHUNK)], dstv)
            pltpu.async_copy(gsrc.at[srcv], rows, sem).wait()
            pltpu.sync_copy(rows, acc.at[dstv], add=True)
            if with_deg:
                pltpu.sync_copy(onesv, dacc.at[dstv], add=True)
            return carry

        lax.fori_loop(0, MAIN_CHUNKS, step, 0)

        toff = base + MAIN_CHUNKS * CHUNK
        pltpu.sync_copy(srcm.at[pl.ds(toff, TAIL)], srct)
        pltpu.sync_copy(dstm.at[pl.ds(toff, TAIL)], dstt)
        pltpu.async_copy(gsrc.at[srct], rowst, sem).wait()
        pltpu.sync_copy(rowst, acc.at[dstt], add=True)
        if with_deg:
            pltpu.sync_copy(onest, dacc.at[dstt], add=True)
        plsc.subcore_barrier()

        _copy_rowrange(sid, acc, part.at[cid])
        if with_deg:
            @pl.when(sid == 0)
            def _():
                pltpu.sync_copy(dacc, degp.at[cid])

    return pl.kernel(body, out_type=out_type, mesh=mesh, scratch_types=scratch)


_sc_agg_deg128 = _make_sc_agg(128, True)
_sc_agg128 = _make_sc_agg(128, False)


def _mm_body(x_ref, w_ref, o1_ref, o2_ref):
    d = jnp.dot(x_ref[...], w_ref[...], preferred_element_type=jnp.float32)
    k = o1_ref.shape[1]
    o1_ref[...] = d[:, :k]
    o2_ref[...] = d[:, k:]


def _layer_body(pa, pb, rd_ref, s_ref, b_ref, w_ref, o1_ref, o2_ref):
    h = jnp.maximum((pa[0] + pb[0]) * rd_ref[...] + b_ref[...] + s_ref[...], 0.0)
    d = jnp.dot(h, w_ref[...], preferred_element_type=jnp.float32)
    k = o1_ref.shape[1]
    o1_ref[...] = d[:, :k]
    o2_ref[...] = d[:, k:]


def _rdeg_body(dt_ref, rd_ref):
    deg = jnp.maximum(dt_ref[...].sum(axis=1, keepdims=True), 1.0)
    rd_ref[...] = 1.0 / deg


def _h_body(pa, pb, rd_ref, s_ref, b_ref, o_ref):
    o_ref[...] = jnp.maximum(
        (pa[0] + pb[0]) * rd_ref[...] + b_ref[...] + s_ref[...], 0.0)


def _final_body(pa, pb, rd_ref, h_ref, b_ref, wl_ref, wr_ref, r_ref, o_ref):
    agg = (pa[0] + pb[0]) * rd_ref[...]
    t = (jnp.dot(agg, wl_ref[...], preferred_element_type=jnp.float32)
         + jnp.dot(h_ref[...], wr_ref[...], preferred_element_type=jnp.float32)
         + b_ref[...])
    h = jax.nn.sigmoid(t)
    contrib = r_ref[...][None, :, :] * h[:, None, :]
    j = lax.broadcasted_iota(jnp.int32, contrib.shape, 2)
    contrib = jnp.where(j < 13, contrib, -jnp.inf)
    o_ref[...] = jnp.max(contrib, axis=2)[:, :13]


def _row_spec(w):
    return pl.BlockSpec((BM, w), lambda i: (i, 0))


def kernel(x, edge_index, Wl0, Wr0, b0, Wl1, Wr1, b1, Wl2, Wr2, b2, R):
    f32 = jnp.float32
    src = edge_index[0]
    dst = edge_index[1]
    zero2_128 = jnp.zeros((N, 128), f32)
    zero1 = jnp.zeros((N,), f32)

    w0 = jnp.concatenate([Wl0.T, Wr0.T], axis=1)
    w1 = jnp.concatenate([Wl1.T, Wr1.T], axis=1)
    wl2p = jnp.pad(Wl2.T, ((0, 0), (0, 3)))
    wr2p = jnp.pad(Wr2.T, ((0, 0), (0, 3)))
    b0r = b0.reshape(1, 128)
    b1r = b1.reshape(1, 128)
    b2r = jnp.pad(b2, (0, 3)).reshape(1, 16)
    Rp = jnp.pad(R, ((0, 3), (0, 3)))

    z0, s0 = pl.pallas_call(
        _mm_body,
        grid=(GRID,),
        in_specs=[_row_spec(128), pl.BlockSpec((128, 256), lambda i: (0, 0))],
        out_specs=[_row_spec(128), _row_spec(128)],
        out_shape=[jax.ShapeDtypeStruct((N, 128), f32)] * 2,
    )(x, w0)

    part0, degp = _sc_agg_deg128(z0, src, dst, zero2_128, zero1)
    degt = degp.T

    rdeg = pl.pallas_call(
        _rdeg_body,
        grid=(GRID,),
        in_specs=[_row_spec(2)],
        out_specs=_row_spec(1),
        out_shape=jax.ShapeDtypeStruct((N, 1), f32),
    )(degt)

    def layer(part, s, br, w, k1, k2):
        return pl.pallas_call(
            _layer_body,
            grid=(GRID,),
            in_specs=[
                pl.BlockSpec((1, BM, 128), lambda i: (0, i, 0)),
                pl.BlockSpec((1, BM, 128), lambda i: (1, i, 0)),
                _row_spec(1),
                _row_spec(128),
                pl.BlockSpec((1, 128), lambda i: (0, 0)),
                pl.BlockSpec(w.shape, lambda i: (0, 0)),
            ],
            out_specs=[_row_spec(k1), _row_spec(k2)],
            out_shape=[jax.ShapeDtypeStruct((N, k1), f32),
                       jax.ShapeDtypeStruct((N, k2), f32)],
        )(part, part, rdeg, s, br, w)

    z1, s1 = layer(part0, s0, b0r, w1, 128, 128)
    part1 = _sc_agg128(z1, src, dst, zero2_128)[0]

    h2 = pl.pallas_call(
        _h_body,
        grid=(GRID,),
        in_specs=[
            pl.BlockSpec((1, BM, 128), lambda i: (0, i, 0)),
            pl.BlockSpec((1, BM, 128), lambda i: (1, i, 0)),
            _row_spec(1),
            _row_spec(128),
            pl.BlockSpec((1, 128), lambda i: (0, 0)),
        ],
        out_specs=_row_spec(128),
        out_shape=jax.ShapeDtypeStruct((N, 128), f32),
    )(part1, part1, rdeg, s1, b1r)
    part2 = _sc_agg128(h2, src, dst, zero2_128)[0]

    out = pl.pallas_call(
        _final_body,
        grid=(GRID,),
        in_specs=[
            pl.BlockSpec((1, BM, 128), lambda i: (0, i, 0)),
            pl.BlockSpec((1, BM, 128), lambda i: (1, i, 0)),
            _row_spec(1),
            _row_spec(128),
            pl.BlockSpec((1, 16), lambda i: (0, 0)),
            pl.BlockSpec((128, 16), lambda i: (0, 0)),
            pl.BlockSpec((128, 16), lambda i: (0, 0)),
            pl.BlockSpec((16, 16), lambda i: (0, 0)),
        ],
        out_specs=_row_spec(13),
        out_shape=jax.ShapeDtypeStruct((N, 13), f32),
    )(part2, part2, rdeg, h2, b2r, wl2p, wr2p, Rp)
    return out


# speedup vs baseline: 12.3918x; 1.9482x over previous
import functools

import jax
import jax.numpy as jnp
from jax import lax
from jax.experimental import pallas as pl
from jax.experimental.pallas import tpu as pltpu
from jax.experimental.pallas import tpu_sc as plsc

N = 10000
E = 320000
CHUNK = 128
NWORK = 32
CPW = 80
E_PAD = NWORK * CPW * CHUNK
NJUNK = 8
SUB_ROWS = 632
LAST_ROWS = N - 15 * SUB_ROWS
BM = 2000
GRID = N // BM


def _copy_rowrange(sid, src_ref, dst_ref):
    @pl.when(sid < 15)
    def _():
        pltpu.sync_copy(src_ref.at[pl.ds(sid * SUB_ROWS, SUB_ROWS)],
                        dst_ref.at[pl.ds(sid * SUB_ROWS, SUB_ROWS)])

    @pl.when(sid == 15)
    def _():
        pltpu.sync_copy(src_ref.at[pl.ds(15 * SUB_ROWS, LAST_ROWS)],
                        dst_ref.at[pl.ds(15 * SUB_ROWS, LAST_ROWS)])


def _make_sc_agg(W: int, with_deg: bool):
    mesh = plsc.VectorSubcoreMesh(core_axis_name="c", subcore_axis_name="s")
    out_type = [jax.ShapeDtypeStruct((2, N, W), jnp.float32)]
    scratch = [
        pltpu.VMEM((CHUNK,), jnp.int32),
        pltpu.VMEM((CHUNK,), jnp.int32),
        pltpu.VMEM((CPW, CHUNK), jnp.int32),
        pltpu.VMEM((CHUNK, W), jnp.float32),
        pltpu.VMEM((CHUNK, W), jnp.float32),
        pltpu.VMEM_SHARED((N + NJUNK, W), jnp.float32),
        pltpu.SemaphoreType.DMA,
        pltpu.SemaphoreType.DMA,
        pltpu.SemaphoreType.DMA,
        pltpu.SemaphoreType.DMA,
    ]
    if with_deg:
        out_type.append(jax.ShapeDtypeStruct((2, N + NJUNK), jnp.float32))
        scratch += [
            pltpu.VMEM((CHUNK,), jnp.float32),
            pltpu.VMEM_SHARED((N + NJUNK,), jnp.float32),
        ]

    def body(z, srcm, dstm, zero2, *rest):
        if with_deg:
            (zero1, part, degp, srcb0, srcb1, dstv, rows0, rows1, acc,
             isem0, isem1, gsem0, gsem1, onesv, dacc) = rest
        else:
            (part, srcb0, srcb1, dstv, rows0, rows1, acc,
             isem0, isem1, gsem0, gsem1) = rest
        srcb = (srcb0, srcb1)
        isem = (isem0, isem1)
        rows = (rows0, rows1)
        gsem = (gsem0, gsem1)
        cid = lax.axis_index("c")
        sid = lax.axis_index("s")
        wid = sid * 2 + cid

        _copy_rowrange(sid, zero2, acc)
        if with_deg:
            @pl.when(sid == 0)
            def _():
                pltpu.sync_copy(zero1, dacc)
            for i in range(CHUNK // 16):
                onesv[pl.ds(i * 16, 16)] = jnp.ones((16,), jnp.float32)

        pltpu.sync_copy(dstm.at[pl.ds(wid * CPW, CPW)], dstv)
        ibase = wid * CPW * CHUNK

        def sstart(g, b):
            pltpu.async_copy(srcm.at[pl.ds(ibase + g * CHUNK, CHUNK)],
                             srcb[b], isem[b])

        def iwait(b):
            pltpu.make_async_copy(srcm.at[pl.ds(0, CHUNK)],
                                  srcb[b], isem[b]).wait()

        def gstart(b):
            pltpu.async_copy(z.at[srcb[b]], rows[b], gsem[b])

        def gwait(b):
            pltpu.make_async_copy(z.at[pl.ds(0, CHUNK)], rows[b], gsem[b]).wait()

        def scat(g, b):
            pltpu.sync_copy(rows[b], acc.at[dstv.at[g]], add=True)
            if with_deg:
                pltpu.sync_copy(onesv, dacc.at[dstv.at[g]], add=True)

        sstart(0, 0)
        sstart(1, 1)
        iwait(0)
        gstart(0)
        plsc.subcore_barrier()

        last_o = CPW // 2 - 1

        def step(o, carry):
            g = o * 2
            iwait(1)
            gstart(1)
            gwait(0)

            @pl.when(o < last_o)
            def _():
                sstart(g + 2, 0)
            scat(g, 0)

            @pl.when(o < last_o)
            def _():
                iwait(0)
                gstart(0)
            gwait(1)

            @pl.when(o < last_o)
            def _():
                sstart(g + 3, 1)
            scat(g + 1, 1)
            return carry

        lax.fori_loop(0, CPW // 2, step, 0)
        plsc.subcore_barrier()

        _copy_rowrange(sid, acc, part.at[cid])
        if with_deg:
            @pl.when(sid == 0)
            def _():
                pltpu.sync_copy(dacc, degp.at[cid])

    return pl.kernel(body, out_type=out_type, mesh=mesh, scratch_types=scratch)


_sc_agg_deg128 = _make_sc_agg(128, True)
_sc_agg128 = _make_sc_agg(128, False)


def _mm_body(x_ref, w_ref, o1_ref, o2_ref):
    d = jnp.dot(x_ref[...], w_ref[...], preferred_element_type=jnp.float32)
    k = o1_ref.shape[1]
    o1_ref[...] = d[:, :k]
    o2_ref[...] = d[:, k:]


def _layer_body(pa, pb, rd_ref, s_ref, b_ref, w_ref, o1_ref, o2_ref):
    h = jnp.maximum((pa[0] + pb[0]) * rd_ref[...] + b_ref[...] + s_ref[...], 0.0)
    d = jnp.dot(h, w_ref[...], preferred_element_type=jnp.float32)
    k = o1_ref.shape[1]
    o1_ref[...] = d[:, :k]
    o2_ref[...] = d[:, k:]


def _rdeg_body(dt_ref, rd_ref):
    deg = jnp.maximum(dt_ref[...].sum(axis=1, keepdims=True), 1.0)
    rd_ref[...] = 1.0 / deg


def _h_body(pa, pb, rd_ref, s_ref, b_ref, o_ref):
    o_ref[...] = jnp.maximum(
        (pa[0] + pb[0]) * rd_ref[...] + b_ref[...] + s_ref[...], 0.0)


def _final_body(pa, pb, rd_ref, h_ref, b_ref, wl_ref, wr_ref, r_ref, o_ref):
    agg = (pa[0] + pb[0]) * rd_ref[...]
    t = (jnp.dot(agg, wl_ref[...], preferred_element_type=jnp.float32)
         + jnp.dot(h_ref[...], wr_ref[...], preferred_element_type=jnp.float32)
         + b_ref[...])
    h = jax.nn.sigmoid(t)
    contrib = r_ref[...][None, :, :] * h[:, None, :]
    j = lax.broadcasted_iota(jnp.int32, contrib.shape, 2)
    contrib = jnp.where(j < 13, contrib, -jnp.inf)
    o_ref[...] = jnp.max(contrib, axis=2)[:, :13]


def _row_spec(w):
    return pl.BlockSpec((BM, w), lambda i: (i, 0))


def kernel(x, edge_index, Wl0, Wr0, b0, Wl1, Wr1, b1, Wl2, Wr2, b2, R):
    f32 = jnp.float32
    npad = E_PAD - E
    pad_iota = jnp.arange(npad, dtype=jnp.int32)
    src = jnp.concatenate([edge_index[0], pad_iota % N])
    dst = jnp.concatenate([edge_index[1], N + pad_iota % NJUNK]).reshape(-1, CHUNK)
    zero2_128 = jnp.zeros((N, 128), f32)
    zero1 = jnp.zeros((N + NJUNK,), f32)

    w0 = jnp.concatenate([Wl0.T, Wr0.T], axis=1)
    w1 = jnp.concatenate([Wl1.T, Wr1.T], axis=1)
    wl2p = jnp.pad(Wl2.T, ((0, 0), (0, 3)))
    wr2p = jnp.pad(Wr2.T, ((0, 0), (0, 3)))
    b0r = b0.reshape(1, 128)
    b1r = b1.reshape(1, 128)
    b2r = jnp.pad(b2, (0, 3)).reshape(1, 16)
    Rp = jnp.pad(R, ((0, 3), (0, 3)))

    z0, s0 = pl.pallas_call(
        _mm_body,
        grid=(GRID,),
        in_specs=[_row_spec(128), pl.BlockSpec((128, 256), lambda i: (0, 0))],
        out_specs=[_row_spec(128), _row_spec(128)],
        out_shape=[jax.ShapeDtypeStruct((N, 128), f32)] * 2,
    )(x, w0)

    part0, degp = _sc_agg_deg128(z0, src, dst, zero2_128, zero1)
    degt = degp[:, :N].T

    rdeg = pl.pallas_call(
        _rdeg_body,
        grid=(GRID,),
        in_specs=[_row_spec(2)],
        out_specs=_row_spec(1),
        out_shape=jax.ShapeDtypeStruct((N, 1), f32),
    )(degt)

    def layer(part, s, br, w, k1, k2):
        return pl.pallas_call(
            _layer_body,
            grid=(GRID,),
            in_specs=[
                pl.BlockSpec((1, BM, 128), lambda i: (0, i, 0)),
                pl.BlockSpec((1, BM, 128), lambda i: (1, i, 0)),
                _row_spec(1),
                _row_spec(128),
                pl.BlockSpec((1, 128), lambda i: (0, 0)),
                pl.BlockSpec(w.shape, lambda i: (0, 0)),
            ],
            out_specs=[_row_spec(k1), _row_spec(k2)],
            out_shape=[jax.ShapeDtypeStruct((N, k1), f32),
                       jax.ShapeDtypeStruct((N, k2), f32)],
        )(part, part, rdeg, s, br, w)

    z1, s1 = layer(part0, s0, b0r, w1, 128, 128)
    part1 = _sc_agg128(z1, src, dst, zero2_128)[0]

    h2 = pl.pallas_call(
        _h_body,
        grid=(GRID,),
        in_specs=[
            pl.BlockSpec((1, BM, 128), lambda i: (0, i, 0)),
            pl.BlockSpec((1, BM, 128), lambda i: (1, i, 0)),
            _row_spec(1),
            _row_spec(128),
            pl.BlockSpec((1, 128), lambda i: (0, 0)),
        ],
        out_specs=_row_spec(128),
        out_shape=jax.ShapeDtypeStruct((N, 128), f32),
    )(part1, part1, rdeg, s1, b1r)
    part2 = _sc_agg128(h2, src, dst, zero2_128)[0]

    out = pl.pallas_call(
        _final_body,
        grid=(GRID,),
        in_specs=[
            pl.BlockSpec((1, BM, 128), lambda i: (0, i, 0)),
            pl.BlockSpec((1, BM, 128), lambda i: (1, i, 0)),
            _row_spec(1),
            _row_spec(128),
            pl.BlockSpec((1, 16), lambda i: (0, 0)),
            pl.BlockSpec((128, 16), lambda i: (0, 0)),
            pl.BlockSpec((128, 16), lambda i: (0, 0)),
            pl.BlockSpec((16, 16), lambda i: (0, 0)),
        ],
        out_specs=_row_spec(13),
        out_shape=jax.ShapeDtypeStruct((N, 13), f32),
    )(part2, part2, rdeg, h2, b2r, wl2p, wr2p, Rp)
    return out


# speedup vs baseline: 13.4839x; 1.0881x over previous
import functools

import jax
import jax.numpy as jnp
from jax import lax
from jax.experimental import pallas as pl
from jax.experimental.pallas import tpu as pltpu
from jax.experimental.pallas import tpu_sc as plsc

N = 10000
E = 320000
CHUNK = 128
NWORK = 32
CPW = 80
E_PAD = NWORK * CPW * CHUNK
NJUNK = 8
SUB_ROWS = 632
LAST_ROWS = N - 15 * SUB_ROWS
BM = 2000
GRID = N // BM


def _copy_rowrange(sid, src_ref, dst_ref):
    @pl.when(sid < 15)
    def _():
        pltpu.sync_copy(src_ref.at[pl.ds(sid * SUB_ROWS, SUB_ROWS)],
                        dst_ref.at[pl.ds(sid * SUB_ROWS, SUB_ROWS)])

    @pl.when(sid == 15)
    def _():
        pltpu.sync_copy(src_ref.at[pl.ds(15 * SUB_ROWS, LAST_ROWS)],
                        dst_ref.at[pl.ds(15 * SUB_ROWS, LAST_ROWS)])


def _make_sc_agg(W: int, with_deg: bool, tc_tiling: bool = True):
    mesh = plsc.VectorSubcoreMesh(core_axis_name="c", subcore_axis_name="s")
    out_type = [jax.ShapeDtypeStruct((2, N, W), jnp.float32)]
    scratch = [
        pltpu.VMEM((CHUNK,), jnp.int32),
        pltpu.VMEM((CHUNK,), jnp.int32),
        pltpu.VMEM((CPW, CHUNK), jnp.int32),
        pltpu.VMEM((CHUNK, W), jnp.float32),
        pltpu.VMEM((CHUNK, W), jnp.float32),
        pltpu.VMEM_SHARED((N + NJUNK, W), jnp.float32),
        pltpu.SemaphoreType.DMA,
        pltpu.SemaphoreType.DMA,
        pltpu.SemaphoreType.DMA,
        pltpu.SemaphoreType.DMA,
    ]
    if with_deg:
        out_type.append(jax.ShapeDtypeStruct((2, N + NJUNK), jnp.float32))
        scratch += [
            pltpu.VMEM((CHUNK,), jnp.float32),
            pltpu.VMEM_SHARED((N + NJUNK,), jnp.float32),
        ]

    def body(z, srcm, dstm, zero2, *rest):
        if with_deg:
            (zero1, part, degp, srcb0, srcb1, dstv, rows0, rows1, acc,
             isem0, isem1, gsem0, gsem1, onesv, dacc) = rest
        else:
            (part, srcb0, srcb1, dstv, rows0, rows1, acc,
             isem0, isem1, gsem0, gsem1) = rest
        srcb = (srcb0, srcb1)
        isem = (isem0, isem1)
        rows = (rows0, rows1)
        gsem = (gsem0, gsem1)
        cid = lax.axis_index("c")
        sid = lax.axis_index("s")
        wid = sid * 2 + cid

        _copy_rowrange(sid, zero2, acc)
        if with_deg:
            @pl.when(sid == 0)
            def _():
                pltpu.sync_copy(zero1, dacc)
            for i in range(CHUNK // 16):
                onesv[pl.ds(i * 16, 16)] = jnp.ones((16,), jnp.float32)

        pltpu.sync_copy(dstm.at[pl.ds(wid * CPW, CPW)], dstv)
        ibase = wid * CPW * CHUNK

        def sstart(g, b):
            pltpu.async_copy(srcm.at[pl.ds(ibase + g * CHUNK, CHUNK)],
                             srcb[b], isem[b])

        def iwait(b):
            pltpu.make_async_copy(srcm.at[pl.ds(0, CHUNK)],
                                  srcb[b], isem[b]).wait()

        def gstart(b):
            pltpu.async_copy(z.at[srcb[b]], rows[b], gsem[b])

        def gwait(b):
            pltpu.make_async_copy(z.at[pl.ds(0, CHUNK)], rows[b], gsem[b]).wait()

        def scat(g, b):
            pltpu.sync_copy(rows[b], acc.at[dstv.at[g]], add=True)
            if with_deg:
                pltpu.sync_copy(onesv, dacc.at[dstv.at[g]], add=True)

        sstart(0, 0)
        sstart(1, 1)
        iwait(0)
        gstart(0)
        plsc.subcore_barrier()

        last_o = CPW // 2 - 1

        def step(o, carry):
            g = o * 2
            iwait(1)
            gstart(1)
            gwait(0)

            @pl.when(o < last_o)
            def _():
                sstart(g + 2, 0)
            scat(g, 0)

            @pl.when(o < last_o)
            def _():
                iwait(0)
                gstart(0)
            gwait(1)

            @pl.when(o < last_o)
            def _():
                sstart(g + 3, 1)
            scat(g + 1, 1)
            return carry

        lax.fori_loop(0, CPW // 2, step, 0)
        plsc.subcore_barrier()

        _copy_rowrange(sid, acc, part.at[cid])
        if with_deg:
            @pl.when(sid == 0)
            def _():
                pltpu.sync_copy(dacc, degp.at[cid])

    params = None if tc_tiling else pltpu.CompilerParams(use_tc_tiling_on_sc=False)
    return pl.kernel(body, out_type=out_type, mesh=mesh, scratch_types=scratch,
                     compiler_params=params)


_sc_agg_deg128 = _make_sc_agg(128, True)
_sc_agg128 = _make_sc_agg(128, False)
_sc_agg16 = _make_sc_agg(16, False, tc_tiling=False)


def _mm_body(x_ref, w_ref, o1_ref, o2_ref):
    d = jnp.dot(x_ref[...], w_ref[...], preferred_element_type=jnp.float32)
    k = o1_ref.shape[1]
    o1_ref[...] = d[:, :k]
    o2_ref[...] = d[:, k:]


def _layer_body(pa, pb, rd_ref, s_ref, b_ref, w_ref, o1_ref, o2_ref):
    h = jnp.maximum((pa[0] + pb[0]) * rd_ref[...] + b_ref[...] + s_ref[...], 0.0)
    d = jnp.dot(h, w_ref[...], preferred_element_type=jnp.float32)
    k = o1_ref.shape[1]
    o1_ref[...] = d[:, :k]
    o2_ref[...] = d[:, k:]


def _rdeg_body(dt_ref, rd_ref):
    deg = jnp.maximum(dt_ref[...].sum(axis=1, keepdims=True), 1.0)
    rd_ref[...] = 1.0 / deg


def _final_body(pa, pb, rd_ref, s_ref, b_ref, r_ref, o_ref):
    h = jax.nn.sigmoid((pa[0] + pb[0]) * rd_ref[...] + b_ref[...] + s_ref[...])
    contrib = r_ref[...][None, :, :] * h[:, None, :]
    j = lax.broadcasted_iota(jnp.int32, contrib.shape, 2)
    contrib = jnp.where(j < 13, contrib, -jnp.inf)
    o_ref[...] = jnp.max(contrib, axis=2)[:, :13]


def _row_spec(w):
    return pl.BlockSpec((BM, w), lambda i: (i, 0))


def kernel(x, edge_index, Wl0, Wr0, b0, Wl1, Wr1, b1, Wl2, Wr2, b2, R):
    f32 = jnp.float32
    npad = E_PAD - E
    pad_iota = jnp.arange(npad, dtype=jnp.int32)
    src = jnp.concatenate([edge_index[0], pad_iota % N])
    dst = jnp.concatenate([edge_index[1], N + pad_iota % NJUNK]).reshape(-1, CHUNK)
    zero2_128 = jnp.zeros((N, 128), f32)
    zero2_16 = jnp.zeros((N, 16), f32)
    zero1 = jnp.zeros((N + NJUNK,), f32)

    w0 = jnp.concatenate([Wl0.T, Wr0.T], axis=1)
    w1 = jnp.concatenate([Wl1.T, Wr1.T], axis=1)
    wl2p = jnp.pad(Wl2.T, ((0, 0), (0, 3)))
    wr2p = jnp.pad(Wr2.T, ((0, 0), (0, 3)))
    w2 = jnp.concatenate([wl2p, wr2p], axis=1)
    b0r = b0.reshape(1, 128)
    b1r = b1.reshape(1, 128)
    b2r = jnp.pad(b2, (0, 3)).reshape(1, 16)
    Rp = jnp.pad(R, ((0, 3), (0, 3)))

    z0, s0 = pl.pallas_call(
        _mm_body,
        grid=(GRID,),
        in_specs=[_row_spec(128), pl.BlockSpec((128, 256), lambda i: (0, 0))],
        out_specs=[_row_spec(128), _row_spec(128)],
        out_shape=[jax.ShapeDtypeStruct((N, 128), f32)] * 2,
    )(x, w0)

    part0, degp = _sc_agg_deg128(z0, src, dst, zero2_128, zero1)
    degt = degp[:, :N].T

    rdeg = pl.pallas_call(
        _rdeg_body,
        grid=(GRID,),
        in_specs=[_row_spec(2)],
        out_specs=_row_spec(1),
        out_shape=jax.ShapeDtypeStruct((N, 1), f32),
    )(degt)

    def layer(part, s, br, w, k1, k2):
        return pl.pallas_call(
            _layer_body,
            grid=(GRID,),
            in_specs=[
                pl.BlockSpec((1, BM, 128), lambda i: (0, i, 0)),
                pl.BlockSpec((1, BM, 128), lambda i: (1, i, 0)),
                _row_spec(1),
                _row_spec(128),
                pl.BlockSpec((1, 128), lambda i: (0, 0)),
                pl.BlockSpec(w.shape, lambda i: (0, 0)),
            ],
            out_specs=[_row_spec(k1), _row_spec(k2)],
            out_shape=[jax.ShapeDtypeStruct((N, k1), f32),
                       jax.ShapeDtypeStruct((N, k2), f32)],
        )(part, part, rdeg, s, br, w)

    z1, s1 = layer(part0, s0, b0r, w1, 128, 128)
    part1 = _sc_agg128(z1, src, dst, zero2_128)[0]

    z2, s2 = layer(part1, s1, b1r, w2, 16, 16)
    part2 = _sc_agg16(z2, src, dst, zero2_16)[0]

    out = pl.pallas_call(
        _final_body,
        grid=(GRID,),
        in_specs=[
            pl.BlockSpec((1, BM, 16), lambda i: (0, i, 0)),
            pl.BlockSpec((1, BM, 16), lambda i: (1, i, 0)),
            _row_spec(1),
            _row_spec(16),
            pl.BlockSpec((1, 16), lambda i: (0, 0)),
            pl.BlockSpec((16, 16), lambda i: (0, 0)),
        ],
        out_specs=_row_spec(13),
        out_shape=jax.ShapeDtypeStruct((N, 13), f32),
    )(part2, part2, rdeg, s2, b2r, Rp)
    return out


# speedup vs baseline: 14.5965x; 1.0825x over previous
import functools

import jax
import jax.numpy as jnp
from jax import lax
from jax.experimental import pallas as pl
from jax.experimental.pallas import tpu as pltpu
from jax.experimental.pallas import tpu_sc as plsc

N = 10000
E = 320000
CHUNK = 128
NWORK = 32
CPW = 80
E_PAD = NWORK * CPW * CHUNK
NJUNK = 8
SUB_ROWS = 632
LAST_ROWS = N - 15 * SUB_ROWS
BM = 2000
GRID = N // BM


def _copy_rowrange(sid, src_ref, dst_ref):
    @pl.when(sid < 15)
    def _():
        pltpu.sync_copy(src_ref.at[pl.ds(sid * SUB_ROWS, SUB_ROWS)],
                        dst_ref.at[pl.ds(sid * SUB_ROWS, SUB_ROWS)])

    @pl.when(sid == 15)
    def _():
        pltpu.sync_copy(src_ref.at[pl.ds(15 * SUB_ROWS, LAST_ROWS)],
                        dst_ref.at[pl.ds(15 * SUB_ROWS, LAST_ROWS)])


def _make_sc_agg_deep(W: int, tc_tiling: bool):
    mesh = plsc.VectorSubcoreMesh(core_axis_name="c", subcore_axis_name="s")
    out_type = jax.ShapeDtypeStruct((2, N, W), jnp.float32)
    NB = 4
    scratch = (
        [pltpu.VMEM((CHUNK,), jnp.int32) for _ in range(NB)]
        + [pltpu.VMEM((CPW, CHUNK), jnp.int32)]
        + [pltpu.VMEM((CHUNK, W), jnp.float32) for _ in range(NB)]
        + [pltpu.VMEM_SHARED((N + NJUNK, W), jnp.float32)]
        + [pltpu.SemaphoreType.DMA] * (3 * NB)
    )

    def body(z, srcm, dstm, zero2, part, *rest):
        srcb = rest[0:NB]
        dstv = rest[NB]
        rows = rest[NB + 1:2 * NB + 1]
        acc = rest[2 * NB + 1]
        isem = rest[2 * NB + 2:2 * NB + 2 + NB]
        gsem = rest[2 * NB + 2 + NB:2 * NB + 2 + 2 * NB]
        ssem = rest[2 * NB + 2 + 2 * NB:2 * NB + 2 + 3 * NB]
        cid = lax.axis_index("c")
        sid = lax.axis_index("s")
        wid = sid * 2 + cid

        _copy_rowrange(sid, zero2, acc)
        pltpu.sync_copy(dstm.at[pl.ds(wid * CPW, CPW)], dstv)
        ibase = wid * CPW * CHUNK

        def sstart(g, b):
            pltpu.async_copy(srcm.at[pl.ds(ibase + g * CHUNK, CHUNK)],
                             srcb[b], isem[b])

        def iwait(b):
            pltpu.make_async_copy(srcm.at[pl.ds(0, CHUNK)],
                                  srcb[b], isem[b]).wait()

        def gstart(b):
            pltpu.async_copy(z.at[srcb[b]], rows[b], gsem[b])

        def gwait(b):
            pltpu.make_async_copy(z.at[pl.ds(0, CHUNK)], rows[b], gsem[b]).wait()

        def sc_start(g, b):
            pltpu.async_copy(rows[b], acc.at[dstv.at[g]], ssem[b], add=True)

        def sc_wait(b):
            pltpu.make_async_copy(rows[b], acc.at[dstv.at[0]], ssem[b]).wait()

        for b in range(NB):
            sstart(b, b)
        iwait(0)
        gstart(0)
        iwait(1)
        gstart(1)
        plsc.subcore_barrier()

        def step(o, carry):
            for k in range(NB):
                g = o * NB + k
                b = k
                b2 = (k + 2) % NB
                always_x = k < 2

                def xblk(need_swait):
                    if need_swait:
                        sc_wait(b2)
                    iwait(b2)
                    gstart(b2)

                if always_x:
                    @pl.when(o > 0)
                    def _():
                        xblk(True)

                    @pl.when(o == 0)
                    def _():
                        xblk(False)
                else:
                    @pl.when(o < CPW // NB - 1)
                    def _():
                        xblk(True)
                gwait(b)

                @pl.when(o < CPW // NB - 1)
                def _():
                    sstart(o * NB + k + NB, b)
                sc_start(g, b)
            return carry

        lax.fori_loop(0, CPW // NB, step, 0)
        for b in range(NB):
            sc_wait(b)
        plsc.subcore_barrier()
        _copy_rowrange(sid, acc, part.at[cid])

    params = None if tc_tiling else pltpu.CompilerParams(use_tc_tiling_on_sc=False)
    return pl.kernel(body, out_type=out_type, mesh=mesh, scratch_types=scratch,
                     compiler_params=params)


def _make_sc_agg(W: int, with_deg: bool, tc_tiling: bool = True):
    mesh = plsc.VectorSubcoreMesh(core_axis_name="c", subcore_axis_name="s")
    out_type = [jax.ShapeDtypeStruct((2, N, W), jnp.float32)]
    scratch = [
        pltpu.VMEM((CHUNK,), jnp.int32),
        pltpu.VMEM((CHUNK,), jnp.int32),
        pltpu.VMEM((CPW, CHUNK), jnp.int32),
        pltpu.VMEM((CHUNK, W), jnp.float32),
        pltpu.VMEM((CHUNK, W), jnp.float32),
        pltpu.VMEM_SHARED((N + NJUNK, W), jnp.float32),
        pltpu.SemaphoreType.DMA,
        pltpu.SemaphoreType.DMA,
        pltpu.SemaphoreType.DMA,
        pltpu.SemaphoreType.DMA,
    ]
    if with_deg:
        out_type.append(jax.ShapeDtypeStruct((2, N + NJUNK), jnp.float32))
        scratch += [
            pltpu.VMEM((CHUNK,), jnp.float32),
            pltpu.VMEM_SHARED((N + NJUNK,), jnp.float32),
        ]

    def body(z, srcm, dstm, zero2, *rest):
        if with_deg:
            (zero1, part, degp, srcb0, srcb1, dstv, rows0, rows1, acc,
             isem0, isem1, gsem0, gsem1, onesv, dacc) = rest
        else:
            (part, srcb0, srcb1, dstv, rows0, rows1, acc,
             isem0, isem1, gsem0, gsem1) = rest
        srcb = (srcb0, srcb1)
        isem = (isem0, isem1)
        rows = (rows0, rows1)
        gsem = (gsem0, gsem1)
        cid = lax.axis_index("c")
        sid = lax.axis_index("s")
        wid = sid * 2 + cid

        _copy_rowrange(sid, zero2, acc)
        if with_deg:
            @pl.when(sid == 0)
            def _():
                pltpu.sync_copy(zero1, dacc)
            for i in range(CHUNK // 16):
                onesv[pl.ds(i * 16, 16)] = jnp.ones((16,), jnp.float32)

        pltpu.sync_copy(dstm.at[pl.ds(wid * CPW, CPW)], dstv)
        ibase = wid * CPW * CHUNK

        def sstart(g, b):
            pltpu.async_copy(srcm.at[pl.ds(ibase + g * CHUNK, CHUNK)],
                             srcb[b], isem[b])

        def iwait(b):
            pltpu.make_async_copy(srcm.at[pl.ds(0, CHUNK)],
                                  srcb[b], isem[b]).wait()

        def gstart(b):
            pltpu.async_copy(z.at[srcb[b]], rows[b], gsem[b])

        def gwait(b):
            pltpu.make_async_copy(z.at[pl.ds(0, CHUNK)], rows[b], gsem[b]).wait()

        def scat(g, b):
            pltpu.sync_copy(rows[b], acc.at[dstv.at[g]], add=True)
            if with_deg:
                pltpu.sync_copy(onesv, dacc.at[dstv.at[g]], add=True)

        sstart(0, 0)
        sstart(1, 1)
        iwait(0)
        gstart(0)
        plsc.subcore_barrier()

        last_o = CPW // 2 - 1

        def step(o, carry):
            g = o * 2
            iwait(1)
            gstart(1)
            gwait(0)

            @pl.when(o < last_o)
            def _():
                sstart(g + 2, 0)
            scat(g, 0)

            @pl.when(o < last_o)
            def _():
                iwait(0)
                gstart(0)
            gwait(1)

            @pl.when(o < last_o)
            def _():
                sstart(g + 3, 1)
            scat(g + 1, 1)
            return carry

        lax.fori_loop(0, CPW // 2, step, 0)
        plsc.subcore_barrier()

        _copy_rowrange(sid, acc, part.at[cid])
        if with_deg:
            @pl.when(sid == 0)
            def _():
                pltpu.sync_copy(dacc, degp.at[cid])

    params = None if tc_tiling else pltpu.CompilerParams(use_tc_tiling_on_sc=False)
    return pl.kernel(body, out_type=out_type, mesh=mesh, scratch_types=scratch,
                     compiler_params=params)


_sc_agg_deg128 = _make_sc_agg(128, True)
_sc_agg128 = _make_sc_agg(128, False)
_sc_agg16 = _make_sc_agg_deep(16, tc_tiling=False)


def _mm_body(x_ref, w_ref, o1_ref, o2_ref):
    d = jnp.dot(x_ref[...], w_ref[...], preferred_element_type=jnp.float32)
    k = o1_ref.shape[1]
    o1_ref[...] = d[:, :k]
    o2_ref[...] = d[:, k:]


def _layer_body(pa, pb, rd_ref, s_ref, b_ref, w_ref, o1_ref, o2_ref):
    h = jnp.maximum((pa[0] + pb[0]) * rd_ref[...] + b_ref[...] + s_ref[...], 0.0)
    d = jnp.dot(h, w_ref[...], preferred_element_type=jnp.float32)
    k = o1_ref.shape[1]
    o1_ref[...] = d[:, :k]
    o2_ref[...] = d[:, k:]


def _layer1_body(pa, pb, dt_ref, s_ref, b_ref, w_ref, o1_ref, o2_ref, rd_ref):
    r = 1.0 / jnp.maximum(dt_ref[...].sum(axis=1, keepdims=True), 1.0)
    h = jnp.maximum((pa[0] + pb[0]) * r + b_ref[...] + s_ref[...], 0.0)
    d = jnp.dot(h, w_ref[...], preferred_element_type=jnp.float32)
    k = o1_ref.shape[1]
    o1_ref[...] = d[:, :k]
    o2_ref[...] = d[:, k:]
    rd_ref[...] = r


def _final_body(pa, pb, rd_ref, s_ref, b_ref, r_ref, o_ref):
    h = jax.nn.sigmoid((pa[0] + pb[0]) * rd_ref[...] + b_ref[...] + s_ref[...])
    contrib = r_ref[...][None, :, :] * h[:, None, :]
    j = lax.broadcasted_iota(jnp.int32, contrib.shape, 2)
    contrib = jnp.where(j < 13, contrib, -jnp.inf)
    o_ref[...] = jnp.max(contrib, axis=2)[:, :13]


def _row_spec(w):
    return pl.BlockSpec((BM, w), lambda i: (i, 0))


def kernel(x, edge_index, Wl0, Wr0, b0, Wl1, Wr1, b1, Wl2, Wr2, b2, R):
    f32 = jnp.float32
    npad = E_PAD - E
    pad_iota = jnp.arange(npad, dtype=jnp.int32)
    src = jnp.concatenate([edge_index[0], pad_iota % N])
    dst = jnp.concatenate([edge_index[1], N + pad_iota % NJUNK]).reshape(-1, CHUNK)
    zero2_128 = jnp.zeros((N, 128), f32)
    zero2_16 = jnp.zeros((N, 16), f32)
    zero1 = jnp.zeros((N + NJUNK,), f32)

    w0 = jnp.concatenate([Wl0.T, Wr0.T], axis=1)
    w1 = jnp.concatenate([Wl1.T, Wr1.T], axis=1)
    wl2p = jnp.pad(Wl2.T, ((0, 0), (0, 3)))
    wr2p = jnp.pad(Wr2.T, ((0, 0), (0, 3)))
    w2 = jnp.concatenate([wl2p, wr2p], axis=1)
    b0r = b0.reshape(1, 128)
    b1r = b1.reshape(1, 128)
    b2r = jnp.pad(b2, (0, 3)).reshape(1, 16)
    Rp = jnp.pad(R, ((0, 3), (0, 3)))

    z0, s0 = pl.pallas_call(
        _mm_body,
        grid=(GRID,),
        in_specs=[_row_spec(128), pl.BlockSpec((128, 256), lambda i: (0, 0))],
        out_specs=[_row_spec(128), _row_spec(128)],
        out_shape=[jax.ShapeDtypeStruct((N, 128), f32)] * 2,
    )(x, w0)

    part0, degp = _sc_agg_deg128(z0, src, dst, zero2_128, zero1)
    degt = degp[:, :N].T

    z1, s1, rdeg = pl.pallas_call(
        _layer1_body,
        grid=(GRID,),
        in_specs=[
            pl.BlockSpec((1, BM, 128), lambda i: (0, i, 0)),
            pl.BlockSpec((1, BM, 128), lambda i: (1, i, 0)),
            _row_spec(2),
            _row_spec(128),
            pl.BlockSpec((1, 128), lambda i: (0, 0)),
            pl.BlockSpec((128, 256), lambda i: (0, 0)),
        ],
        out_specs=[_row_spec(128), _row_spec(128), _row_spec(1)],
        out_shape=[jax.ShapeDtypeStruct((N, 128), f32),
                   jax.ShapeDtypeStruct((N, 128), f32),
                   jax.ShapeDtypeStruct((N, 1), f32)],
    )(part0, part0, degt, s0, b0r, w1)
    part1 = _sc_agg128(z1, src, dst, zero2_128)[0]

    z2, s2 = pl.pallas_call(
        _layer_body,
        grid=(GRID,),
        in_specs=[
            pl.BlockSpec((1, BM, 128), lambda i: (0, i, 0)),
            pl.BlockSpec((1, BM, 128), lambda i: (1, i, 0)),
            _row_spec(1),
            _row_spec(128),
            pl.BlockSpec((1, 128), lambda i: (0, 0)),
            pl.BlockSpec((128, 32), lambda i: (0, 0)),
        ],
        out_specs=[_row_spec(16), _row_spec(16)],
        out_shape=[jax.ShapeDtypeStruct((N, 16), f32),
                   jax.ShapeDtypeStruct((N, 16), f32)],
    )(part1, part1, rdeg, s1, b1r, w2)
    part2 = _sc_agg16(z2, src, dst, zero2_16)

    out = pl.pallas_call(
        _final_body,
        grid=(GRID,),
        in_specs=[
            pl.BlockSpec((1, BM, 16), lambda i: (0, i, 0)),
            pl.BlockSpec((1, BM, 16), lambda i: (1, i, 0)),
            _row_spec(1),
            _row_spec(16),
            pl.BlockSpec((1, 16), lambda i: (0, 0)),
            pl.BlockSpec((16, 16), lambda i: (0, 0)),
        ],
        out_specs=_row_spec(13),
        out_shape=jax.ShapeDtypeStruct((N, 13), f32),
    )(part2, part2, rdeg, s2, b2r, Rp)
    return out


# speedup vs baseline: 14.6932x; 1.0066x over previous
import functools

import jax
import jax.numpy as jnp
from jax import lax
from jax.experimental import pallas as pl
from jax.experimental.pallas import tpu as pltpu
from jax.experimental.pallas import tpu_sc as plsc

N = 10000
E = 320000
CHUNK = 128
NWORK = 32
CPW = 80
E_PAD = NWORK * CPW * CHUNK
NJUNK = 8
SUB_ROWS = 632
LAST_ROWS = N - 15 * SUB_ROWS
BM = 2000
GRID = N // BM


def _copy_rowrange(sid, src_ref, dst_ref):
    @pl.when(sid < 15)
    def _():
        pltpu.sync_copy(src_ref.at[pl.ds(sid * SUB_ROWS, SUB_ROWS)],
                        dst_ref.at[pl.ds(sid * SUB_ROWS, SUB_ROWS)])

    @pl.when(sid == 15)
    def _():
        pltpu.sync_copy(src_ref.at[pl.ds(15 * SUB_ROWS, LAST_ROWS)],
                        dst_ref.at[pl.ds(15 * SUB_ROWS, LAST_ROWS)])


def _make_sc_agg_deep(W: int, tc_tiling: bool):
    mesh = plsc.VectorSubcoreMesh(core_axis_name="c", subcore_axis_name="s")
    out_type = jax.ShapeDtypeStruct((2, N, W), jnp.float32)
    NB = 4
    scratch = (
        [pltpu.VMEM((CHUNK,), jnp.int32) for _ in range(NB)]
        + [pltpu.VMEM((CPW, CHUNK), jnp.int32)]
        + [pltpu.VMEM((CHUNK, W), jnp.float32) for _ in range(NB)]
        + [pltpu.VMEM_SHARED((N + NJUNK, W), jnp.float32)]
        + [pltpu.SemaphoreType.DMA] * (3 * NB)
    )

    def body(z, srcm, dstm, zero2, part, *rest):
        srcb = rest[0:NB]
        dstv = rest[NB]
        rows = rest[NB + 1:2 * NB + 1]
        acc = rest[2 * NB + 1]
        isem = rest[2 * NB + 2:2 * NB + 2 + NB]
        gsem = rest[2 * NB + 2 + NB:2 * NB + 2 + 2 * NB]
        ssem = rest[2 * NB + 2 + 2 * NB:2 * NB + 2 + 3 * NB]
        cid = lax.axis_index("c")
        sid = lax.axis_index("s")
        wid = sid * 2 + cid

        pltpu.sync_copy(dstm.at[pl.ds(wid * CPW, CPW)], dstv)
        ibase = wid * CPW * CHUNK

        def sstart(g, b):
            pltpu.async_copy(srcm.at[pl.ds(ibase + g * CHUNK, CHUNK)],
                             srcb[b], isem[b])

        def iwait(b):
            pltpu.make_async_copy(srcm.at[pl.ds(0, CHUNK)],
                                  srcb[b], isem[b]).wait()

        def gstart(b):
            pltpu.async_copy(z.at[srcb[b]], rows[b], gsem[b])

        def gwait(b):
            pltpu.make_async_copy(z.at[pl.ds(0, CHUNK)], rows[b], gsem[b]).wait()

        def sc_start(g, b):
            pltpu.async_copy(rows[b], acc.at[dstv.at[g]], ssem[b], add=True)

        def sc_wait(b):
            pltpu.make_async_copy(rows[b], acc.at[dstv.at[0]], ssem[b]).wait()

        for b in range(NB):
            sstart(b, b)
        iwait(0)
        gstart(0)
        iwait(1)
        gstart(1)
        _copy_rowrange(sid, zero2, acc)
        plsc.subcore_barrier()

        def step(o, carry):
            for k in range(NB):
                g = o * NB + k
                b = k
                b2 = (k + 2) % NB
                always_x = k < 2

                def xblk(need_swait):
                    if need_swait:
                        sc_wait(b2)
                    iwait(b2)
                    gstart(b2)

                if always_x:
                    @pl.when(o > 0)
                    def _():
                        xblk(True)

                    @pl.when(o == 0)
                    def _():
                        xblk(False)
                else:
                    @pl.when(o < CPW // NB - 1)
                    def _():
                        xblk(True)
                gwait(b)

                @pl.when(o < CPW // NB - 1)
                def _():
                    sstart(o * NB + k + NB, b)
                sc_start(g, b)
            return carry

        lax.fori_loop(0, CPW // NB, step, 0)
        for b in range(NB):
            sc_wait(b)
        plsc.subcore_barrier()
        _copy_rowrange(sid, acc, part.at[cid])

    params = None if tc_tiling else pltpu.CompilerParams(use_tc_tiling_on_sc=False)
    return pl.kernel(body, out_type=out_type, mesh=mesh, scratch_types=scratch,
                     compiler_params=params)


def _make_sc_agg(W: int, with_deg: bool, tc_tiling: bool = True):
    mesh = plsc.VectorSubcoreMesh(core_axis_name="c", subcore_axis_name="s")
    out_type = [jax.ShapeDtypeStruct((2, N, W), jnp.float32)]
    scratch = [
        pltpu.VMEM((CHUNK,), jnp.int32),
        pltpu.VMEM((CHUNK,), jnp.int32),
        pltpu.VMEM((CPW, CHUNK), jnp.int32),
        pltpu.VMEM((CHUNK, W), jnp.float32),
        pltpu.VMEM((CHUNK, W), jnp.float32),
        pltpu.VMEM_SHARED((N + NJUNK, W), jnp.float32),
        pltpu.SemaphoreType.DMA,
        pltpu.SemaphoreType.DMA,
        pltpu.SemaphoreType.DMA,
        pltpu.SemaphoreType.DMA,
    ]
    if with_deg:
        out_type.append(jax.ShapeDtypeStruct((2, N + NJUNK), jnp.float32))
        scratch += [
            pltpu.VMEM((CHUNK,), jnp.float32),
            pltpu.VMEM_SHARED((N + NJUNK,), jnp.float32),
        ]

    def body(z, srcm, dstm, zero2, *rest):
        if with_deg:
            (zero1, part, degp, srcb0, srcb1, dstv, rows0, rows1, acc,
             isem0, isem1, gsem0, gsem1, onesv, dacc) = rest
        else:
            (part, srcb0, srcb1, dstv, rows0, rows1, acc,
             isem0, isem1, gsem0, gsem1) = rest
        srcb = (srcb0, srcb1)
        isem = (isem0, isem1)
        rows = (rows0, rows1)
        gsem = (gsem0, gsem1)
        cid = lax.axis_index("c")
        sid = lax.axis_index("s")
        wid = sid * 2 + cid

        pltpu.sync_copy(dstm.at[pl.ds(wid * CPW, CPW)], dstv)
        ibase = wid * CPW * CHUNK

        def sstart(g, b):
            pltpu.async_copy(srcm.at[pl.ds(ibase + g * CHUNK, CHUNK)],
                             srcb[b], isem[b])

        def iwait(b):
            pltpu.make_async_copy(srcm.at[pl.ds(0, CHUNK)],
                                  srcb[b], isem[b]).wait()

        def gstart(b):
            pltpu.async_copy(z.at[srcb[b]], rows[b], gsem[b])

        def gwait(b):
            pltpu.make_async_copy(z.at[pl.ds(0, CHUNK)], rows[b], gsem[b]).wait()

        def scat(g, b):
            pltpu.sync_copy(rows[b], acc.at[dstv.at[g]], add=True)
            if with_deg:
                pltpu.sync_copy(onesv, dacc.at[dstv.at[g]], add=True)

        sstart(0, 0)
        sstart(1, 1)
        iwait(0)
        gstart(0)
        _copy_rowrange(sid, zero2, acc)
        if with_deg:
            @pl.when(sid == 0)
            def _():
                pltpu.sync_copy(zero1, dacc)
            for i in range(CHUNK // 16):
                onesv[pl.ds(i * 16, 16)] = jnp.ones((16,), jnp.float32)
        plsc.subcore_barrier()

        last_o = CPW // 2 - 1

        def step(o, carry):
            g = o * 2
            iwait(1)
            gstart(1)
            gwait(0)

            @pl.when(o < last_o)
            def _():
                sstart(g + 2, 0)
            scat(g, 0)

            @pl.when(o < last_o)
            def _():
                iwait(0)
                gstart(0)
            gwait(1)

            @pl.when(o < last_o)
            def _():
                sstart(g + 3, 1)
            scat(g + 1, 1)
            return carry

        lax.fori_loop(0, CPW // 2, step, 0)
        plsc.subcore_barrier()

        _copy_rowrange(sid, acc, part.at[cid])
        if with_deg:
            @pl.when(sid == 0)
            def _():
                pltpu.sync_copy(dacc, degp.at[cid])

    params = None if tc_tiling else pltpu.CompilerParams(use_tc_tiling_on_sc=False)
    return pl.kernel(body, out_type=out_type, mesh=mesh, scratch_types=scratch,
                     compiler_params=params)


_sc_agg_deg128 = _make_sc_agg(128, True)
_sc_agg128 = _make_sc_agg(128, False)
_sc_agg16 = _make_sc_agg_deep(16, tc_tiling=False)


def _mm_body(x_ref, w_ref, o1_ref, o2_ref):
    d = jnp.dot(x_ref[...], w_ref[...], preferred_element_type=jnp.float32)
    k = o1_ref.shape[1]
    o1_ref[...] = d[:, :k]
    o2_ref[...] = d[:, k:]


def _layer_body(pa, pb, rd_ref, s_ref, b_ref, w_ref, o1_ref, o2_ref):
    h = jnp.maximum((pa[0] + pb[0]) * rd_ref[...] + b_ref[...] + s_ref[...], 0.0)
    d = jnp.dot(h, w_ref[...], preferred_element_type=jnp.float32)
    k = o1_ref.shape[1]
    o1_ref[...] = d[:, :k]
    o2_ref[...] = d[:, k:]


def _layer1_body(pa, pb, dt_ref, s_ref, b_ref, w_ref, o1_ref, o2_ref, rd_ref):
    r = 1.0 / jnp.maximum(dt_ref[...].sum(axis=1, keepdims=True), 1.0)
    h = jnp.maximum((pa[0] + pb[0]) * r + b_ref[...] + s_ref[...], 0.0)
    d = jnp.dot(h, w_ref[...], preferred_element_type=jnp.float32)
    k = o1_ref.shape[1]
    o1_ref[...] = d[:, :k]
    o2_ref[...] = d[:, k:]
    rd_ref[...] = r


def _final_body(pa, pb, rd_ref, s_ref, b_ref, r_ref, o_ref):
    h = jax.nn.sigmoid((pa[0] + pb[0]) * rd_ref[...] + b_ref[...] + s_ref[...])
    contrib = r_ref[...][None, :, :] * h[:, None, :]
    j = lax.broadcasted_iota(jnp.int32, contrib.shape, 2)
    contrib = jnp.where(j < 13, contrib, -jnp.inf)
    o_ref[...] = jnp.max(contrib, axis=2)[:, :13]


def _row_spec(w):
    return pl.BlockSpec((BM, w), lambda i: (i, 0))


def kernel(x, edge_index, Wl0, Wr0, b0, Wl1, Wr1, b1, Wl2, Wr2, b2, R):
    f32 = jnp.float32
    npad = E_PAD - E
    pad_iota = jnp.arange(npad, dtype=jnp.int32)
    src = jnp.concatenate([edge_index[0], pad_iota % N])
    dst = jnp.concatenate([edge_index[1], N + pad_iota % NJUNK]).reshape(-1, CHUNK)
    zero2_128 = jnp.zeros((N, 128), f32)
    zero2_16 = jnp.zeros((N, 16), f32)
    zero1 = jnp.zeros((N + NJUNK,), f32)

    w0 = jnp.concatenate([Wl0.T, Wr0.T], axis=1)
    w1 = jnp.concatenate([Wl1.T, Wr1.T], axis=1)
    wl2p = jnp.pad(Wl2.T, ((0, 0), (0, 3)))
    wr2p = jnp.pad(Wr2.T, ((0, 0), (0, 3)))
    w2 = jnp.concatenate([wl2p, wr2p], axis=1)
    b0r = b0.reshape(1, 128)
    b1r = b1.reshape(1, 128)
    b2r = jnp.pad(b2, (0, 3)).reshape(1, 16)
    Rp = jnp.pad(R, ((0, 3), (0, 3)))

    z0, s0 = pl.pallas_call(
        _mm_body,
        grid=(GRID,),
        in_specs=[_row_spec(128), pl.BlockSpec((128, 256), lambda i: (0, 0))],
        out_specs=[_row_spec(128), _row_spec(128)],
        out_shape=[jax.ShapeDtypeStruct((N, 128), f32)] * 2,
    )(x, w0)

    part0, degp = _sc_agg_deg128(z0, src, dst, zero2_128, zero1)
    degt = degp[:, :N].T

    z1, s1, rdeg = pl.pallas_call(
        _layer1_body,
        grid=(GRID,),
        in_specs=[
            pl.BlockSpec((1, BM, 128), lambda i: (0, i, 0)),
            pl.BlockSpec((1, BM, 128), lambda i: (1, i, 0)),
            _row_spec(2),
            _row_spec(128),
            pl.BlockSpec((1, 128), lambda i: (0, 0)),
            pl.BlockSpec((128, 256), lambda i: (0, 0)),
        ],
        out_specs=[_row_spec(128), _row_spec(128), _row_spec(1)],
        out_shape=[jax.ShapeDtypeStruct((N, 128), f32),
                   jax.ShapeDtypeStruct((N, 128), f32),
                   jax.ShapeDtypeStruct((N, 1), f32)],
    )(part0, part0, degt, s0, b0r, w1)
    part1 = _sc_agg128(z1, src, dst, zero2_128)[0]

    z2, s2 = pl.pallas_call(
        _layer_body,
        grid=(GRID,),
        in_specs=[
            pl.BlockSpec((1, BM, 128), lambda i: (0, i, 0)),
            pl.BlockSpec((1, BM, 128), lambda i: (1, i, 0)),
            _row_spec(1),
            _row_spec(128),
            pl.BlockSpec((1, 128), lambda i: (0, 0)),
            pl.BlockSpec((128, 32), lambda i: (0, 0)),
        ],
        out_specs=[_row_spec(16), _row_spec(16)],
        out_shape=[jax.ShapeDtypeStruct((N, 16), f32),
                   jax.ShapeDtypeStruct((N, 16), f32)],
    )(part1, part1, rdeg, s1, b1r, w2)
    part2 = _sc_agg16(z2, src, dst, zero2_16)

    out = pl.pallas_call(
        _final_body,
        grid=(GRID,),
        in_specs=[
            pl.BlockSpec((1, BM, 16), lambda i: (0, i, 0)),
            pl.BlockSpec((1, BM, 16), lambda i: (1, i, 0)),
            _row_spec(1),
            _row_spec(16),
            pl.BlockSpec((1, 16), lambda i: (0, 0)),
            pl.BlockSpec((16, 16), lambda i: (0, 0)),
        ],
        out_specs=_row_spec(13),
        out_shape=jax.ShapeDtypeStruct((N, 13), f32),
    )(part2, part2, rdeg, s2, b2r, Rp)
    return out


# speedup vs baseline: 15.2650x; 1.0389x over previous
import functools

import jax
import jax.numpy as jnp
from jax import lax
from jax.experimental import pallas as pl
from jax.experimental.pallas import tpu as pltpu
from jax.experimental.pallas import tpu_sc as plsc

N = 10000
E = 320000
CHUNK = 128
NWORK = 32
CPW = 80
E_PAD = NWORK * CPW * CHUNK
NJUNK = 8
SUB_ROWS = 632
LAST_ROWS = N - 15 * SUB_ROWS
BM = 2000
GRID = N // BM


def _copy_rowrange(sid, src_ref, dst_ref):
    @pl.when(sid < 15)
    def _():
        pltpu.sync_copy(src_ref.at[pl.ds(sid * SUB_ROWS, SUB_ROWS)],
                        dst_ref.at[pl.ds(sid * SUB_ROWS, SUB_ROWS)])

    @pl.when(sid == 15)
    def _():
        pltpu.sync_copy(src_ref.at[pl.ds(15 * SUB_ROWS, LAST_ROWS)],
                        dst_ref.at[pl.ds(15 * SUB_ROWS, LAST_ROWS)])


def _make_sc_agg_deep(W: int, tc_tiling: bool, ch: int = CHUNK,
                      with_deg: bool = False):
    mesh = plsc.VectorSubcoreMesh(core_axis_name="c", subcore_axis_name="s")
    NB = 4
    cpw = (E_PAD // NWORK) // ch
    assert cpw % NB == 0
    out_type = [jax.ShapeDtypeStruct((2, N, W), jnp.float32)]
    scratch = (
        [pltpu.VMEM((ch,), jnp.int32) for _ in range(NB)]
        + [pltpu.VMEM((ch,), jnp.int32) for _ in range(NB)]
        + [pltpu.VMEM((ch, W), jnp.float32) for _ in range(NB)]
        + [pltpu.VMEM_SHARED((N + NJUNK, W), jnp.float32)]
        + [pltpu.SemaphoreType.DMA] * (4 * NB)
    )
    if with_deg:
        out_type.append(jax.ShapeDtypeStruct((2, N + NJUNK), jnp.float32))
        scratch += (
            [pltpu.VMEM((ch,), jnp.float32)]
            + [pltpu.VMEM_SHARED((N + NJUNK,), jnp.float32)]
            + [pltpu.SemaphoreType.DMA] * NB
        )

    def body(z, srcm, dstm, zero2, *rest):
        if with_deg:
            zero1 = rest[0]
            rest = rest[1:]
        part = rest[0]
        if with_deg:
            degp = rest[1]
            rest = rest[2:]
        else:
            rest = rest[1:]
        srcb = rest[0:NB]
        dstb = rest[NB:2 * NB]
        rows = rest[2 * NB:3 * NB]
        acc = rest[3 * NB]
        sems = rest[3 * NB + 1:3 * NB + 1 + 4 * NB]
        isem = sems[0:NB]
        jsem = sems[NB:2 * NB]
        gsem = sems[2 * NB:3 * NB]
        ssem = sems[3 * NB:4 * NB]
        if with_deg:
            onesv = rest[3 * NB + 1 + 4 * NB]
            dacc = rest[3 * NB + 2 + 4 * NB]
            dsem = rest[3 * NB + 3 + 4 * NB:3 * NB + 3 + 5 * NB]
        cid = lax.axis_index("c")
        sid = lax.axis_index("s")
        wid = sid * 2 + cid
        ibase = wid * cpw * ch

        def sstart(g, b):
            pltpu.async_copy(srcm.at[pl.ds(ibase + g * ch, ch)],
                             srcb[b], isem[b])

        def iwait(b):
            pltpu.make_async_copy(srcm.at[pl.ds(0, ch)],
                                  srcb[b], isem[b]).wait()

        def dstart(g, b):
            pltpu.async_copy(dstm.at[pl.ds(ibase + g * ch, ch)],
                             dstb[b], jsem[b])

        def dwait(b):
            pltpu.make_async_copy(dstm.at[pl.ds(0, ch)],
                                  dstb[b], jsem[b]).wait()

        def gstart(b):
            pltpu.async_copy(z.at[srcb[b]], rows[b], gsem[b])

        def gwait(b):
            pltpu.make_async_copy(z.at[pl.ds(0, ch)], rows[b], gsem[b]).wait()

        def sc_start(b):
            pltpu.async_copy(rows[b], acc.at[dstb[b]], ssem[b], add=True)
            if with_deg:
                pltpu.async_copy(onesv, dacc.at[dstb[b]], dsem[b], add=True)

        def sc_wait(b):
            pltpu.make_async_copy(rows[b], acc.at[dstb[b]], ssem[b]).wait()
            if with_deg:
                pltpu.make_async_copy(onesv, dacc.at[dstb[b]], dsem[b]).wait()

        for b in range(NB):
            sstart(b, b)
        dstart(0, 0)
        dstart(1, 1)
        iwait(0)
        gstart(0)
        iwait(1)
        gstart(1)
        _copy_rowrange(sid, zero2, acc)
        if with_deg:
            @pl.when(sid == 0)
            def _():
                pltpu.sync_copy(zero1, dacc)
            for i in range(ch // 16):
                onesv[pl.ds(i * 16, 16)] = jnp.ones((16,), jnp.float32)
        plsc.subcore_barrier()

        def step(o, carry):
            for k in range(NB):
                g = o * NB + k
                b = k
                b2 = (k + 2) % NB
                always_x = k < 2

                def xblk(need_swait):
                    if need_swait:
                        sc_wait(b2)
                    dstart(g + 2, b2)
                    iwait(b2)
                    gstart(b2)

                if always_x:
                    @pl.when(o > 0)
                    def _():
                        xblk(True)

                    @pl.when(o == 0)
                    def _():
                        xblk(False)
                else:
                    @pl.when(o < cpw // NB - 1)
                    def _():
                        xblk(True)
                gwait(b)

                @pl.when(o < cpw // NB - 1)
                def _():
                    sstart(o * NB + k + NB, b)
                dwait(b)
                sc_start(b)
            return carry

        lax.fori_loop(0, cpw // NB, step, 0)
        for b in range(NB):
            sc_wait(b)
        plsc.subcore_barrier()
        _copy_rowrange(sid, acc, part.at[cid])
        if with_deg:
            @pl.when(sid == 0)
            def _():
                pltpu.sync_copy(dacc, degp.at[cid])

    params = None if tc_tiling else pltpu.CompilerParams(use_tc_tiling_on_sc=False)
    return pl.kernel(body, out_type=out_type, mesh=mesh, scratch_types=scratch,
                     compiler_params=params)


def _make_sc_agg(W: int, with_deg: bool, tc_tiling: bool = True):
    mesh = plsc.VectorSubcoreMesh(core_axis_name="c", subcore_axis_name="s")
    out_type = [jax.ShapeDtypeStruct((2, N, W), jnp.float32)]
    scratch = [
        pltpu.VMEM((CHUNK,), jnp.int32),
        pltpu.VMEM((CHUNK,), jnp.int32),
        pltpu.VMEM((CPW, CHUNK), jnp.int32),
        pltpu.VMEM((CHUNK, W), jnp.float32),
        pltpu.VMEM((CHUNK, W), jnp.float32),
        pltpu.VMEM_SHARED((N + NJUNK, W), jnp.float32),
        pltpu.SemaphoreType.DMA,
        pltpu.SemaphoreType.DMA,
        pltpu.SemaphoreType.DMA,
        pltpu.SemaphoreType.DMA,
    ]
    if with_deg:
        out_type.append(jax.ShapeDtypeStruct((2, N + NJUNK), jnp.float32))
        scratch += [
            pltpu.VMEM((CHUNK,), jnp.float32),
            pltpu.VMEM_SHARED((N + NJUNK,), jnp.float32),
        ]

    def body(z, srcm, dstm, zero2, *rest):
        if with_deg:
            (zero1, part, degp, srcb0, srcb1, dstv, rows0, rows1, acc,
             isem0, isem1, gsem0, gsem1, onesv, dacc) = rest
        else:
            (part, srcb0, srcb1, dstv, rows0, rows1, acc,
             isem0, isem1, gsem0, gsem1) = rest
        srcb = (srcb0, srcb1)
        isem = (isem0, isem1)
        rows = (rows0, rows1)
        gsem = (gsem0, gsem1)
        cid = lax.axis_index("c")
        sid = lax.axis_index("s")
        wid = sid * 2 + cid

        pltpu.sync_copy(dstm.at[pl.ds(wid * CPW, CPW)], dstv)
        ibase = wid * CPW * CHUNK

        def sstart(g, b):
            pltpu.async_copy(srcm.at[pl.ds(ibase + g * CHUNK, CHUNK)],
                             srcb[b], isem[b])

        def iwait(b):
            pltpu.make_async_copy(srcm.at[pl.ds(0, CHUNK)],
                                  srcb[b], isem[b]).wait()

        def gstart(b):
            pltpu.async_copy(z.at[srcb[b]], rows[b], gsem[b])

        def gwait(b):
            pltpu.make_async_copy(z.at[pl.ds(0, CHUNK)], rows[b], gsem[b]).wait()

        def scat(g, b):
            pltpu.sync_copy(rows[b], acc.at[dstv.at[g]], add=True)
            if with_deg:
                pltpu.sync_copy(onesv, dacc.at[dstv.at[g]], add=True)

        sstart(0, 0)
        sstart(1, 1)
        iwait(0)
        gstart(0)
        _copy_rowrange(sid, zero2, acc)
        if with_deg:
            @pl.when(sid == 0)
            def _():
                pltpu.sync_copy(zero1, dacc)
            for i in range(CHUNK // 16):
                onesv[pl.ds(i * 16, 16)] = jnp.ones((16,), jnp.float32)
        plsc.subcore_barrier()

        last_o = CPW // 2 - 1

        def step(o, carry):
            g = o * 2
            iwait(1)
            gstart(1)
            gwait(0)

            @pl.when(o < last_o)
            def _():
                sstart(g + 2, 0)
            scat(g, 0)

            @pl.when(o < last_o)
            def _():
                iwait(0)
                gstart(0)
            gwait(1)

            @pl.when(o < last_o)
            def _():
                sstart(g + 3, 1)
            scat(g + 1, 1)
            return carry

        lax.fori_loop(0, CPW // 2, step, 0)
        plsc.subcore_barrier()

        _copy_rowrange(sid, acc, part.at[cid])
        if with_deg:
            @pl.when(sid == 0)
            def _():
                pltpu.sync_copy(dacc, degp.at[cid])

    params = None if tc_tiling else pltpu.CompilerParams(use_tc_tiling_on_sc=False)
    return pl.kernel(body, out_type=out_type, mesh=mesh, scratch_types=scratch,
                     compiler_params=params)


_sc_agg_deg128 = _make_sc_agg_deep(128, True, ch=64, with_deg=True)
_sc_agg128 = _make_sc_agg_deep(128, True, ch=64)
_sc_agg16 = _make_sc_agg_deep(16, tc_tiling=False)


def _mm_body(x_ref, w_ref, o1_ref, o2_ref):
    d = jnp.dot(x_ref[...], w_ref[...], preferred_element_type=jnp.float32)
    k = o1_ref.shape[1]
    o1_ref[...] = d[:, :k]
    o2_ref[...] = d[:, k:]


def _layer_body(pa, pb, rd_ref, s_ref, b_ref, w_ref, o1_ref, o2_ref):
    h = jnp.maximum((pa[0] + pb[0]) * rd_ref[...] + b_ref[...] + s_ref[...], 0.0)
    d = jnp.dot(h, w_ref[...], preferred_element_type=jnp.float32)
    k = o1_ref.shape[1]
    o1_ref[...] = d[:, :k]
    o2_ref[...] = d[:, k:]


def _layer1_body(pa, pb, dt_ref, s_ref, b_ref, w_ref, o1_ref, o2_ref, rd_ref):
    r = 1.0 / jnp.maximum(dt_ref[...].sum(axis=1, keepdims=True), 1.0)
    h = jnp.maximum((pa[0] + pb[0]) * r + b_ref[...] + s_ref[...], 0.0)
    d = jnp.dot(h, w_ref[...], preferred_element_type=jnp.float32)
    k = o1_ref.shape[1]
    o1_ref[...] = d[:, :k]
    o2_ref[...] = d[:, k:]
    rd_ref[...] = r


def _final_body(pa, pb, rd_ref, s_ref, b_ref, r_ref, o_ref):
    h = jax.nn.sigmoid((pa[0] + pb[0]) * rd_ref[...] + b_ref[...] + s_ref[...])
    contrib = r_ref[...][None, :, :] * h[:, None, :]
    j = lax.broadcasted_iota(jnp.int32, contrib.shape, 2)
    contrib = jnp.where(j < 13, contrib, -jnp.inf)
    o_ref[...] = jnp.max(contrib, axis=2)[:, :13]


def _row_spec(w):
    return pl.BlockSpec((BM, w), lambda i: (i, 0))


def kernel(x, edge_index, Wl0, Wr0, b0, Wl1, Wr1, b1, Wl2, Wr2, b2, R):
    f32 = jnp.float32
    npad = E_PAD - E
    pad_iota = jnp.arange(npad, dtype=jnp.int32)
    src = jnp.concatenate([edge_index[0], pad_iota % N])
    dst = jnp.concatenate([edge_index[1], N + pad_iota % NJUNK])
    zero2_128 = jnp.zeros((N, 128), f32)
    zero2_16 = jnp.zeros((N, 16), f32)
    zero1 = jnp.zeros((N + NJUNK,), f32)

    w0 = jnp.concatenate([Wl0.T, Wr0.T], axis=1)
    w1 = jnp.concatenate([Wl1.T, Wr1.T], axis=1)
    wl2p = jnp.pad(Wl2.T, ((0, 0), (0, 3)))
    wr2p = jnp.pad(Wr2.T, ((0, 0), (0, 3)))
    w2 = jnp.concatenate([wl2p, wr2p], axis=1)
    b0r = b0.reshape(1, 128)
    b1r = b1.reshape(1, 128)
    b2r = jnp.pad(b2, (0, 3)).reshape(1, 16)
    Rp = jnp.pad(R, ((0, 3), (0, 3)))

    z0, s0 = pl.pallas_call(
        _mm_body,
        grid=(GRID,),
        in_specs=[_row_spec(128), pl.BlockSpec((128, 256), lambda i: (0, 0))],
        out_specs=[_row_spec(128), _row_spec(128)],
        out_shape=[jax.ShapeDtypeStruct((N, 128), f32)] * 2,
    )(x, w0)

    part0, degp = _sc_agg_deg128(z0, src, dst, zero2_128, zero1)
    degt = degp[:, :N].T

    z1, s1, rdeg = pl.pallas_call(
        _layer1_body,
        grid=(GRID,),
        in_specs=[
            pl.BlockSpec((1, BM, 128), lambda i: (0, i, 0)),
            pl.BlockSpec((1, BM, 128), lambda i: (1, i, 0)),
            _row_spec(2),
            _row_spec(128),
            pl.BlockSpec((1, 128), lambda i: (0, 0)),
            pl.BlockSpec((128, 256), lambda i: (0, 0)),
        ],
        out_specs=[_row_spec(128), _row_spec(128), _row_spec(1)],
        out_shape=[jax.ShapeDtypeStruct((N, 128), f32),
                   jax.ShapeDtypeStruct((N, 128), f32),
                   jax.ShapeDtypeStruct((N, 1), f32)],
    )(part0, part0, degt, s0, b0r, w1)
    part1 = _sc_agg128(z1, src, dst, zero2_128)[0]

    z2, s2 = pl.pallas_call(
        _layer_body,
        grid=(GRID,),
        in_specs=[
            pl.BlockSpec((1, BM, 128), lambda i: (0, i, 0)),
            pl.BlockSpec((1, BM, 128), lambda i: (1, i, 0)),
            _row_spec(1),
            _row_spec(128),
            pl.BlockSpec((1, 128), lambda i: (0, 0)),
            pl.BlockSpec((128, 32), lambda i: (0, 0)),
        ],
        out_specs=[_row_spec(16), _row_spec(16)],
        out_shape=[jax.ShapeDtypeStruct((N, 16), f32),
                   jax.ShapeDtypeStruct((N, 16), f32)],
    )(part1, part1, rdeg, s1, b1r, w2)
    part2 = _sc_agg16(z2, src, dst, zero2_16)[0]

    out = pl.pallas_call(
        _final_body,
        grid=(GRID,),
        in_specs=[
            pl.BlockSpec((1, BM, 16), lambda i: (0, i, 0)),
            pl.BlockSpec((1, BM, 16), lambda i: (1, i, 0)),
            _row_spec(1),
            _row_spec(16),
            pl.BlockSpec((1, 16), lambda i: (0, 0)),
            pl.BlockSpec((16, 16), lambda i: (0, 0)),
        ],
        out_specs=_row_spec(13),
        out_shape=jax.ShapeDtypeStruct((N, 13), f32),
    )(part2, part2, rdeg, s2, b2r, Rp)
    return out
